# Initial kernel scaffold; baseline (speedup 1.0000x reference)
import jax
import jax.numpy as jnp
from jax.experimental import pallas as pl


def kernel(input, emb_vals, main_vals, emb_rows, emb_cols, main_rows, main_cols):
    raise NotImplementedError("write your pallas kernel here")


# speedup vs baseline: 12.0235x; 12.0235x over previous
import functools

import jax
import jax.numpy as jnp
from jax import lax
from jax.experimental import pallas as pl
from jax.experimental.pallas import tpu as pltpu
from jax.experimental.pallas import tpu_sc as plsc

_B = 1024
_D_IN = 1024
_N_EMB = 512
_D_OUT = 1024
_D_CAT = _D_IN + _N_EMB

_NS = 16
_L = 16


def _make_densify(nnz, n_rows, n_cols):
  chunk = nnz // _NS
  nchunk = chunk // 128
  n_el = n_rows * n_cols
  zlen = 8192
  zslice = n_el // _NS
  olen = n_el // (2 * _NS)
  assert chunk % (8 * _L) == 0 and nchunk * 128 == chunk
  assert zslice % zlen == 0 and olen % 8 == 0

  mesh = plsc.VectorSubcoreMesh(core_axis_name="c", subcore_axis_name="s")

  @functools.partial(
      pl.kernel,
      mesh=mesh,
      out_type=jax.ShapeDtypeStruct((n_el,), jnp.float32),
      scratch_types=[
          pltpu.VMEM((chunk,), jnp.int32),
          pltpu.VMEM((chunk,), jnp.int32),
          pltpu.VMEM((chunk,), jnp.float32),
          pltpu.VMEM((nchunk, 128), jnp.int32),
          pltpu.VMEM((zlen,), jnp.float32),
          pltpu.VMEM_SHARED((n_el,), jnp.float32),
      ],
  )
  def dens(rows_hbm, cols_hbm, vals_hbm, out_hbm,
           rows_v, cols_v, vals_v, idx_v, z_v, acc_s):
    cid = lax.axis_index("c")
    sid = lax.axis_index("s")
    wid = cid * _NS + sid
    base = sid * chunk

    pltpu.sync_copy(rows_hbm.at[pl.ds(base, chunk)], rows_v)
    pltpu.sync_copy(cols_hbm.at[pl.ds(base, chunk)], cols_v)
    pltpu.sync_copy(vals_hbm.at[pl.ds(base, chunk)], vals_v)

    zvec = jnp.zeros((_L,), jnp.float32)

    def zfill(i, carry):
      for u in range(8):
        z_v[pl.ds((i * 8 + u) * _L, _L)] = zvec
      return carry

    lax.fori_loop(0, zlen // (8 * _L), zfill, 0)

    def zdma(i, carry):
      pltpu.sync_copy(z_v, acc_s.at[pl.ds(sid * zslice + i * zlen, zlen)])
      return carry

    lax.fori_loop(0, zslice // zlen, zdma, 0)

    def build(k, carry):
      for u in range(8):
        j = k * 8 + u
        sl = pl.ds(j * _L, _L)
        flat = rows_v[sl] * n_cols + cols_v[sl]
        idx_v[k, pl.ds(u * _L, _L)] = flat
      return carry

    lax.fori_loop(0, nchunk, build, 0)

    plsc.subcore_barrier()

    def scat(k, carry):
      pltpu.sync_copy(vals_v.at[pl.ds(k * 128, 128)],
                      acc_s.at[idx_v.at[k]], add=True)
      return carry

    lax.fori_loop(0, nchunk, scat, 0)

    plsc.subcore_barrier()

    pltpu.sync_copy(acc_s.at[pl.ds(wid * olen, olen)],
                    out_hbm.at[pl.ds(wid * olen, olen)])

  return dens


_densify_emb = _make_densify(8192, _N_EMB, _D_IN)
_densify_main = _make_densify(32768, _D_OUT, _D_CAT)


def _tc_body(x_ref, e_ref, m_ref, o_ref):
  xb = x_ref[...]
  emb = lax.dot_general(xb, e_ref[...], (((1,), (1,)), ((), ())),
                        preferred_element_type=jnp.float32)
  cat = jnp.concatenate([xb, emb], axis=1)
  o_ref[...] = lax.dot_general(cat, m_ref[...], (((1,), (1,)), ((), ())),
                               preferred_element_type=jnp.float32)


_BB = 256


def _tc_forward(x, e_dense, m_dense):
  return pl.pallas_call(
      _tc_body,
      grid=(_B // _BB,),
      in_specs=[
          pl.BlockSpec((_BB, _D_IN), lambda i: (i, 0)),
          pl.BlockSpec((_N_EMB, _D_IN), lambda i: (0, 0)),
          pl.BlockSpec((_D_OUT, _D_CAT), lambda i: (0, 0)),
      ],
      out_specs=pl.BlockSpec((_BB, _D_OUT), lambda i: (i, 0)),
      out_shape=jax.ShapeDtypeStruct((_B, _D_OUT), jnp.float32),
  )(x, e_dense, m_dense)


def kernel(input, emb_vals, main_vals, emb_rows, emb_cols, main_rows, main_cols):
  e16 = _densify_emb(emb_rows.astype(jnp.int32), emb_cols.astype(jnp.int32),
                     emb_vals)
  m16 = _densify_main(main_rows.astype(jnp.int32), main_cols.astype(jnp.int32),
                      main_vals)
  e = e16.reshape(_N_EMB, _D_IN)
  m = m16.reshape(_D_OUT, _D_CAT)
  return _tc_forward(input, e, m)


# speedup vs baseline: 12.2629x; 1.0199x over previous
import functools

import jax
import jax.numpy as jnp
from jax import lax
from jax.experimental import pallas as pl
from jax.experimental.pallas import tpu as pltpu
from jax.experimental.pallas import tpu_sc as plsc

_B = 1024
_D_IN = 1024
_N_EMB = 512
_D_OUT = 1024
_D_CAT = _D_IN + _N_EMB

_NNZ_E = 8192
_NNZ_M = 32768

_NS = 16
_L = 16
_ZLEN = 8192


def _densify_body(sid, n_cols, n_el, rows_hbm, cols_hbm, vals_hbm, out_hbm,
                  rows_v, cols_v, vals_v, idx_v, z_v, acc_s):
  chunk = rows_v.shape[0]
  nchunk = chunk // 128
  zslice = n_el // _NS
  base = sid * chunk

  pltpu.sync_copy(rows_hbm.at[pl.ds(base, chunk)], rows_v)
  pltpu.sync_copy(cols_hbm.at[pl.ds(base, chunk)], cols_v)
  pltpu.sync_copy(vals_hbm.at[pl.ds(base, chunk)], vals_v)

  def zdma(i, carry):
    pltpu.sync_copy(z_v, acc_s.at[pl.ds(sid * zslice + i * _ZLEN, _ZLEN)])
    return carry

  lax.fori_loop(0, zslice // _ZLEN, zdma, 0)

  def build(k, carry):
    for u in range(8):
      j = k * 8 + u
      sl = pl.ds(j * _L, _L)
      flat = rows_v[sl] * n_cols + cols_v[sl]
      idx_v[k, pl.ds(u * _L, _L)] = flat
    return carry

  lax.fori_loop(0, nchunk, build, 0)

  plsc.subcore_barrier()

  def scat(k, carry):
    pltpu.sync_copy(vals_v.at[pl.ds(k * 128, 128)],
                    acc_s.at[idx_v.at[k]], add=True)
    return carry

  lax.fori_loop(0, nchunk, scat, 0)

  plsc.subcore_barrier()

  olen = n_el // _NS
  pltpu.sync_copy(acc_s.at[pl.ds(sid * olen, olen)],
                  out_hbm.at[pl.ds(sid * olen, olen)])


def _make_densify_both():
  n_el_e = _N_EMB * _D_IN
  n_el_m = _D_OUT * _D_CAT
  chunk_e = _NNZ_E // _NS
  chunk_m = _NNZ_M // _NS

  mesh = plsc.VectorSubcoreMesh(core_axis_name="c", subcore_axis_name="s")

  @functools.partial(
      pl.kernel,
      mesh=mesh,
      out_type=(jax.ShapeDtypeStruct((n_el_e,), jnp.float32),
                jax.ShapeDtypeStruct((n_el_m,), jnp.float32)),
      scratch_types=[
          pltpu.VMEM((chunk_e,), jnp.int32),
          pltpu.VMEM((chunk_e,), jnp.int32),
          pltpu.VMEM((chunk_e,), jnp.float32),
          pltpu.VMEM((chunk_e // 128, 128), jnp.int32),
          pltpu.VMEM((chunk_m,), jnp.int32),
          pltpu.VMEM((chunk_m,), jnp.int32),
          pltpu.VMEM((chunk_m,), jnp.float32),
          pltpu.VMEM((chunk_m // 128, 128), jnp.int32),
          pltpu.VMEM((_ZLEN,), jnp.float32),
          pltpu.VMEM_SHARED((n_el_m,), jnp.float32),
      ],
  )
  def dens(erows_hbm, ecols_hbm, evals_hbm, mrows_hbm, mcols_hbm, mvals_hbm,
           eout_hbm, mout_hbm,
           erows_v, ecols_v, evals_v, eidx_v,
           mrows_v, mcols_v, mvals_v, midx_v, z_v, acc_s):
    cid = lax.axis_index("c")
    sid = lax.axis_index("s")

    zvec = jnp.zeros((_L,), jnp.float32)

    def zfill(i, carry):
      for u in range(8):
        z_v[pl.ds((i * 8 + u) * _L, _L)] = zvec
      return carry

    lax.fori_loop(0, _ZLEN // (8 * _L), zfill, 0)

    @pl.when(cid == 0)
    def _():
      _densify_body(sid, _D_IN, n_el_e, erows_hbm, ecols_hbm, evals_hbm,
                    eout_hbm, erows_v, ecols_v, evals_v, eidx_v, z_v, acc_s)

    @pl.when(cid == 1)
    def _():
      _densify_body(sid, _D_CAT, n_el_m, mrows_hbm, mcols_hbm, mvals_hbm,
                    mout_hbm, mrows_v, mcols_v, mvals_v, midx_v, z_v, acc_s)

  return dens


_densify_both = _make_densify_both()


def _tc_body(x_ref, e_ref, m_ref, o_ref):
  xb = x_ref[...]
  emb = lax.dot_general(xb, e_ref[...], (((1,), (1,)), ((), ())),
                        preferred_element_type=jnp.float32)
  cat = jnp.concatenate([xb, emb], axis=1)
  o_ref[...] = lax.dot_general(cat, m_ref[...], (((1,), (1,)), ((), ())),
                               preferred_element_type=jnp.float32)


_BB = 256


def _tc_forward(x, e_dense, m_dense):
  return pl.pallas_call(
      _tc_body,
      grid=(_B // _BB,),
      in_specs=[
          pl.BlockSpec((_BB, _D_IN), lambda i: (i, 0)),
          pl.BlockSpec((_N_EMB, _D_IN), lambda i: (0, 0)),
          pl.BlockSpec((_D_OUT, _D_CAT), lambda i: (0, 0)),
      ],
      out_specs=pl.BlockSpec((_BB, _D_OUT), lambda i: (i, 0)),
      out_shape=jax.ShapeDtypeStruct((_B, _D_OUT), jnp.float32),
  )(x, e_dense, m_dense)


def kernel(input, emb_vals, main_vals, emb_rows, emb_cols, main_rows, main_cols):
  e16, m16 = _densify_both(
      emb_rows.astype(jnp.int32), emb_cols.astype(jnp.int32), emb_vals,
      main_rows.astype(jnp.int32), main_cols.astype(jnp.int32), main_vals)
  e = e16.reshape(_N_EMB, _D_IN)
  m = m16.reshape(_D_OUT, _D_CAT)
  return _tc_forward(input, e, m)


# speedup vs baseline: 15.0342x; 1.2260x over previous
import functools

import jax
import jax.numpy as jnp
from jax import lax
from jax.experimental import pallas as pl
from jax.experimental.pallas import tpu as pltpu
from jax.experimental.pallas import tpu_sc as plsc

_B = 1024
_D_IN = 1024
_N_EMB = 512
_D_OUT = 1024
_D_CAT = _D_IN + _N_EMB

_NNZ_E = 8192
_NNZ_M = 32768

_NS = 16
_L = 16
_ZLEN = 8192


def _densify_body(sid, n_cols, n_el, rows_hbm, cols_hbm, vals_hbm, out_hbm,
                  rows_v, cols_v, vals_v, idx_v, z_v, acc_s, sem):
  chunk = rows_v.shape[0]
  nchunk = chunk // 128
  zslice = n_el // _NS
  base = sid * chunk

  pltpu.sync_copy(rows_hbm.at[pl.ds(base, chunk)], rows_v)
  pltpu.sync_copy(cols_hbm.at[pl.ds(base, chunk)], cols_v)
  pltpu.sync_copy(vals_hbm.at[pl.ds(base, chunk)], vals_v)

  def zdma(i, carry):
    pltpu.sync_copy(z_v, acc_s.at[pl.ds(sid * zslice + i * _ZLEN, _ZLEN)])
    return carry

  lax.fori_loop(0, zslice // _ZLEN, zdma, 0)

  def build(k, carry):
    for u in range(8):
      j = k * 8 + u
      sl = pl.ds(j * _L, _L)
      flat = rows_v[sl] * n_cols + cols_v[sl]
      idx_v[k, pl.ds(u * _L, _L)] = flat
    return carry

  lax.fori_loop(0, nchunk, build, 0)

  plsc.subcore_barrier()

  def scat(k, carry):
    pltpu.sync_copy(vals_v.at[pl.ds(k * 128, 128)],
                    acc_s.at[idx_v.at[k]], add=True)
    return carry

  lax.fori_loop(0, nchunk, scat, 0)

  plsc.subcore_barrier()

  n_rows = n_el // n_cols
  rpt = n_rows // _NS
  base_row = sid * rpt

  def ofire(r, carry):
    row = base_row + r
    pltpu.async_copy(acc_s.at[pl.ds(row * n_cols, n_cols)],
                     out_hbm.at[row], sem)
    return carry

  lax.fori_loop(0, rpt, ofire, 0)

  def odrain(r, carry):
    pltpu.make_async_copy(out_hbm.at[base_row + r],
                          out_hbm.at[base_row + r], sem).wait()
    return carry

  lax.fori_loop(0, rpt, odrain, 0)


def _make_densify_both():
  n_el_e = _N_EMB * _D_IN
  n_el_m = _D_OUT * _D_CAT
  chunk_e = _NNZ_E // _NS
  chunk_m = _NNZ_M // _NS

  mesh = plsc.VectorSubcoreMesh(core_axis_name="c", subcore_axis_name="s")

  @functools.partial(
      pl.kernel,
      mesh=mesh,
      out_type=(jax.ShapeDtypeStruct((_N_EMB, _D_IN), jnp.float32),
                jax.ShapeDtypeStruct((_D_OUT, _D_CAT), jnp.float32)),
      scratch_types=[
          pltpu.VMEM((chunk_e,), jnp.int32),
          pltpu.VMEM((chunk_e,), jnp.int32),
          pltpu.VMEM((chunk_e,), jnp.float32),
          pltpu.VMEM((chunk_e // 128, 128), jnp.int32),
          pltpu.VMEM((chunk_m,), jnp.int32),
          pltpu.VMEM((chunk_m,), jnp.int32),
          pltpu.VMEM((chunk_m,), jnp.float32),
          pltpu.VMEM((chunk_m // 128, 128), jnp.int32),
          pltpu.VMEM((_ZLEN,), jnp.float32),
          pltpu.VMEM_SHARED((n_el_m,), jnp.float32),
          pltpu.SemaphoreType.DMA,
      ],
  )
  def dens(erows_hbm, ecols_hbm, evals_hbm, mrows_hbm, mcols_hbm, mvals_hbm,
           eout_hbm, mout_hbm,
           erows_v, ecols_v, evals_v, eidx_v,
           mrows_v, mcols_v, mvals_v, midx_v, z_v, acc_s, sem):
    cid = lax.axis_index("c")
    sid = lax.axis_index("s")

    zvec = jnp.zeros((_L,), jnp.float32)

    def zfill(i, carry):
      for u in range(8):
        z_v[pl.ds((i * 8 + u) * _L, _L)] = zvec
      return carry

    lax.fori_loop(0, _ZLEN // (8 * _L), zfill, 0)

    @pl.when(cid == 0)
    def _():
      _densify_body(sid, _D_IN, n_el_e, erows_hbm, ecols_hbm, evals_hbm,
                    eout_hbm, erows_v, ecols_v, evals_v, eidx_v, z_v, acc_s,
                    sem)

    @pl.when(cid == 1)
    def _():
      _densify_body(sid, _D_CAT, n_el_m, mrows_hbm, mcols_hbm, mvals_hbm,
                    mout_hbm, mrows_v, mcols_v, mvals_v, midx_v, z_v, acc_s,
                    sem)

  return dens


_densify_both = _make_densify_both()


def _tc_body(x_ref, e_ref, m_ref, o_ref):
  xb = x_ref[...]
  emb = lax.dot_general(xb, e_ref[...], (((1,), (1,)), ((), ())),
                        preferred_element_type=jnp.float32)
  cat = jnp.concatenate([xb, emb], axis=1)
  o_ref[...] = lax.dot_general(cat, m_ref[...], (((1,), (1,)), ((), ())),
                               preferred_element_type=jnp.float32)


_BB = 256


def _tc_forward(x, e_dense, m_dense):
  return pl.pallas_call(
      _tc_body,
      grid=(_B // _BB,),
      in_specs=[
          pl.BlockSpec((_BB, _D_IN), lambda i: (i, 0)),
          pl.BlockSpec((_N_EMB, _D_IN), lambda i: (0, 0)),
          pl.BlockSpec((_D_OUT, _D_CAT), lambda i: (0, 0)),
      ],
      out_specs=pl.BlockSpec((_BB, _D_OUT), lambda i: (i, 0)),
      out_shape=jax.ShapeDtypeStruct((_B, _D_OUT), jnp.float32),
  )(x, e_dense, m_dense)


def kernel(input, emb_vals, main_vals, emb_rows, emb_cols, main_rows, main_cols):
  e, m = _densify_both(
      emb_rows.astype(jnp.int32), emb_cols.astype(jnp.int32), emb_vals,
      main_rows.astype(jnp.int32), main_cols.astype(jnp.int32), main_vals)
  return _tc_forward(input, e, m)


# speedup vs baseline: 15.9751x; 1.0626x over previous
import functools

import jax
import jax.numpy as jnp
from jax import lax
from jax.experimental import pallas as pl
from jax.experimental.pallas import tpu as pltpu
from jax.experimental.pallas import tpu_sc as plsc

_B = 1024
_D_IN = 1024
_N_EMB = 512
_D_OUT = 1024
_D_CAT = _D_IN + _N_EMB

_NNZ_E = 8192
_NNZ_M = 32768

_NS = 16
_L = 16
_ZLEN = 8192


def _densify_body(sid, n_cols, n_el, rows_hbm, cols_hbm, vals_hbm, out_hbm,
                  rows_v, cols_v, vals_v, idx_v, z_v, acc_s, sem, zsem):
  chunk = rows_v.shape[0]
  nchunk = chunk // 128
  zslice = n_el // _NS
  base = sid * chunk

  loads = [
      pltpu.async_copy(rows_hbm.at[pl.ds(base, chunk)], rows_v, sem),
      pltpu.async_copy(cols_hbm.at[pl.ds(base, chunk)], cols_v, sem),
      pltpu.async_copy(vals_hbm.at[pl.ds(base, chunk)], vals_v, sem),
  ]
  zeros = [
      pltpu.async_copy(z_v, acc_s.at[pl.ds(sid * zslice + i * _ZLEN, _ZLEN)],
                       zsem)
      for i in range(zslice // _ZLEN)
  ]
  for h in loads:
    h.wait()

  def build(k, carry):
    for u in range(8):
      j = k * 8 + u
      sl = pl.ds(j * _L, _L)
      flat = rows_v[sl] * n_cols + cols_v[sl]
      idx_v[k, pl.ds(u * _L, _L)] = flat
    return carry

  lax.fori_loop(0, nchunk, build, 0)

  for h in zeros:
    h.wait()

  plsc.subcore_barrier()

  def scat(k, carry):
    pltpu.sync_copy(vals_v.at[pl.ds(k * 128, 128)],
                    acc_s.at[idx_v.at[k]], add=True)
    return carry

  lax.fori_loop(0, nchunk, scat, 0)

  plsc.subcore_barrier()

  n_rows = n_el // n_cols
  rpt = n_rows // _NS
  base_row = sid * rpt

  def ofire(r, carry):
    row = base_row + r
    pltpu.async_copy(acc_s.at[pl.ds(row * n_cols, n_cols)],
                     out_hbm.at[row], sem)
    return carry

  lax.fori_loop(0, rpt, ofire, 0)

  def odrain(r, carry):
    pltpu.make_async_copy(out_hbm.at[base_row + r],
                          out_hbm.at[base_row + r], sem).wait()
    return carry

  lax.fori_loop(0, rpt, odrain, 0)


def _make_densify_both():
  n_el_e = _N_EMB * _D_IN
  n_el_m = _D_OUT * _D_CAT
  chunk_e = _NNZ_E // _NS
  chunk_m = _NNZ_M // _NS

  mesh = plsc.VectorSubcoreMesh(core_axis_name="c", subcore_axis_name="s")

  @functools.partial(
      pl.kernel,
      mesh=mesh,
      out_type=(jax.ShapeDtypeStruct((_N_EMB, _D_IN), jnp.float32),
                jax.ShapeDtypeStruct((_D_OUT, _D_CAT), jnp.float32)),
      scratch_types=[
          pltpu.VMEM((chunk_e,), jnp.int32),
          pltpu.VMEM((chunk_e,), jnp.int32),
          pltpu.VMEM((chunk_e,), jnp.float32),
          pltpu.VMEM((chunk_e // 128, 128), jnp.int32),
          pltpu.VMEM((chunk_m,), jnp.int32),
          pltpu.VMEM((chunk_m,), jnp.int32),
          pltpu.VMEM((chunk_m,), jnp.float32),
          pltpu.VMEM((chunk_m // 128, 128), jnp.int32),
          pltpu.VMEM((_ZLEN,), jnp.float32),
          pltpu.VMEM_SHARED((n_el_m,), jnp.float32),
          pltpu.SemaphoreType.DMA,
          pltpu.SemaphoreType.DMA,
      ],
  )
  def dens(erows_hbm, ecols_hbm, evals_hbm, mrows_hbm, mcols_hbm, mvals_hbm,
           eout_hbm, mout_hbm,
           erows_v, ecols_v, evals_v, eidx_v,
           mrows_v, mcols_v, mvals_v, midx_v, z_v, acc_s, sem, zsem):
    cid = lax.axis_index("c")
    sid = lax.axis_index("s")

    zvec = jnp.zeros((_L,), jnp.float32)

    def zfill(i, carry):
      for u in range(8):
        z_v[pl.ds((i * 8 + u) * _L, _L)] = zvec
      return carry

    lax.fori_loop(0, _ZLEN // (8 * _L), zfill, 0)

    @pl.when(cid == 0)
    def _():
      _densify_body(sid, _D_IN, n_el_e, erows_hbm, ecols_hbm, evals_hbm,
                    eout_hbm, erows_v, ecols_v, evals_v, eidx_v, z_v, acc_s,
                    sem, zsem)

    @pl.when(cid == 1)
    def _():
      _densify_body(sid, _D_CAT, n_el_m, mrows_hbm, mcols_hbm, mvals_hbm,
                    mout_hbm, mrows_v, mcols_v, mvals_v, midx_v, z_v, acc_s,
                    sem, zsem)

  return dens


_densify_both = _make_densify_both()


def _tc_body(x_ref, e_ref, m_ref, o_ref):
  xb = x_ref[...]
  emb = lax.dot_general(xb, e_ref[...], (((1,), (1,)), ((), ())),
                        preferred_element_type=jnp.float32)
  cat = jnp.concatenate([xb, emb], axis=1)
  o_ref[...] = lax.dot_general(cat, m_ref[...], (((1,), (1,)), ((), ())),
                               preferred_element_type=jnp.float32)


_BB = 256


def _tc_forward(x, e_dense, m_dense):
  return pl.pallas_call(
      _tc_body,
      grid=(_B // _BB,),
      in_specs=[
          pl.BlockSpec((_BB, _D_IN), lambda i: (i, 0)),
          pl.BlockSpec((_N_EMB, _D_IN), lambda i: (0, 0)),
          pl.BlockSpec((_D_OUT, _D_CAT), lambda i: (0, 0)),
      ],
      out_specs=pl.BlockSpec((_BB, _D_OUT), lambda i: (i, 0)),
      out_shape=jax.ShapeDtypeStruct((_B, _D_OUT), jnp.float32),
  )(x, e_dense, m_dense)


def kernel(input, emb_vals, main_vals, emb_rows, emb_cols, main_rows, main_cols):
  e, m = _densify_both(
      emb_rows.astype(jnp.int32), emb_cols.astype(jnp.int32), emb_vals,
      main_rows.astype(jnp.int32), main_cols.astype(jnp.int32), main_vals)
  return _tc_forward(input, e, m)


# speedup vs baseline: 16.1218x; 1.0092x over previous
import functools

import jax
import jax.numpy as jnp
from jax import lax
from jax.experimental import pallas as pl
from jax.experimental.pallas import tpu as pltpu
from jax.experimental.pallas import tpu_sc as plsc

_B = 1024
_D_IN = 1024
_N_EMB = 512
_D_OUT = 1024
_D_CAT = _D_IN + _N_EMB

_NNZ_E = 8192
_NNZ_M = 32768

_NS = 16
_L = 16
_ZLEN = 8192

_E_CHUNK = _NNZ_E // _NS
_M_CHUNK = _NNZ_M // _NS
_C_CHUNK = _E_CHUNK + _M_CHUNK
_NCHUNK = _C_CHUNK // 128

_E_HALF_ROWS = _N_EMB // 2
_M_HALF_ROWS = _D_OUT // 2
_E_HALF_EL = _E_HALF_ROWS * _D_IN
_M_HALF_EL = _M_HALF_ROWS * _D_CAT
_M_OFF = _E_HALF_EL
_TRASH = _E_HALF_EL + _M_HALF_EL
_ACC_EL = _TRASH + _L
_ZSLICE = (_E_HALF_EL + _M_HALF_EL) // _NS

_E_ROWS_PER_TILE = _E_HALF_ROWS // _NS
_M_ROWS_PER_TILE = _M_HALF_ROWS // _NS


def _make_densify_both():
  mesh = plsc.VectorSubcoreMesh(core_axis_name="c", subcore_axis_name="s")

  @functools.partial(
      pl.kernel,
      mesh=mesh,
      out_type=(jax.ShapeDtypeStruct((_N_EMB, _D_IN), jnp.float32),
                jax.ShapeDtypeStruct((_D_OUT, _D_CAT), jnp.float32)),
      scratch_types=[
          pltpu.VMEM((_C_CHUNK,), jnp.int32),
          pltpu.VMEM((_C_CHUNK,), jnp.int32),
          pltpu.VMEM((_C_CHUNK,), jnp.float32),
          pltpu.VMEM((_NCHUNK, 128), jnp.int32),
          pltpu.VMEM((_ZLEN,), jnp.float32),
          pltpu.VMEM_SHARED((_ACC_EL,), jnp.float32),
          pltpu.SemaphoreType.DMA,
          pltpu.SemaphoreType.DMA,
      ],
  )
  def dens(erows_hbm, ecols_hbm, evals_hbm, mrows_hbm, mcols_hbm, mvals_hbm,
           eout_hbm, mout_hbm,
           rows_v, cols_v, vals_v, idx_v, z_v, acc_s, sem, zsem):
    cid = lax.axis_index("c")
    sid = lax.axis_index("s")
    ebase = sid * _E_CHUNK
    mbase = sid * _M_CHUNK

    zvec = jnp.zeros((_L,), jnp.float32)

    def zfill(i, carry):
      for u in range(8):
        z_v[pl.ds((i * 8 + u) * _L, _L)] = zvec
      return carry

    lax.fori_loop(0, _ZLEN // (8 * _L), zfill, 0)

    zeros = [
        pltpu.async_copy(
            z_v, acc_s.at[pl.ds(sid * _ZSLICE + i * _ZLEN, _ZLEN)], zsem)
        for i in range(_ZSLICE // _ZLEN)
    ]

    loads = [
        pltpu.async_copy(erows_hbm.at[pl.ds(ebase, _E_CHUNK)],
                         rows_v.at[pl.ds(0, _E_CHUNK)], sem),
        pltpu.async_copy(ecols_hbm.at[pl.ds(ebase, _E_CHUNK)],
                         cols_v.at[pl.ds(0, _E_CHUNK)], sem),
        pltpu.async_copy(evals_hbm.at[pl.ds(ebase, _E_CHUNK)],
                         vals_v.at[pl.ds(0, _E_CHUNK)], sem),
        pltpu.async_copy(mrows_hbm.at[pl.ds(mbase, _M_CHUNK)],
                         rows_v.at[pl.ds(_E_CHUNK, _M_CHUNK)], sem),
        pltpu.async_copy(mcols_hbm.at[pl.ds(mbase, _M_CHUNK)],
                         cols_v.at[pl.ds(_E_CHUNK, _M_CHUNK)], sem),
        pltpu.async_copy(mvals_hbm.at[pl.ds(mbase, _M_CHUNK)],
                         vals_v.at[pl.ds(_E_CHUNK, _M_CHUNK)], sem),
    ]
    for h in loads:
      h.wait()

    iota = lax.iota(jnp.int32, _L)
    e_row0 = cid * _E_HALF_ROWS
    m_row0 = cid * _M_HALF_ROWS

    def build_e(k, carry):
      for u in range(8):
        j = k * 8 + u
        sl = pl.ds(j * _L, _L)
        r = rows_v[sl]
        c = cols_v[sl]
        mine = (r >= e_row0) & (r < e_row0 + _E_HALF_ROWS)
        loc = (r - e_row0) * _D_IN + c
        idx_v[k, pl.ds(u * _L, _L)] = jnp.where(mine, loc, _TRASH + iota)
      return carry

    lax.fori_loop(0, _E_CHUNK // 128, build_e, 0)

    def build_m(k, carry):
      for u in range(8):
        j = k * 8 + u
        sl = pl.ds(_E_CHUNK + j * _L, _L)
        r = rows_v[sl]
        c = cols_v[sl]
        mine = (r >= m_row0) & (r < m_row0 + _M_HALF_ROWS)
        loc = _M_OFF + (r - m_row0) * _D_CAT + c
        idx_v[_E_CHUNK // 128 + k, pl.ds(u * _L, _L)] = (
            jnp.where(mine, loc, _TRASH + iota))
      return carry

    lax.fori_loop(0, _M_CHUNK // 128, build_m, 0)

    for h in zeros:
      h.wait()
    plsc.subcore_barrier()

    def scat(k, carry):
      pltpu.sync_copy(vals_v.at[pl.ds(k * 128, 128)],
                      acc_s.at[idx_v.at[k]], add=True)
      return carry

    lax.fori_loop(0, _NCHUNK, scat, 0)

    plsc.subcore_barrier()

    copies = []
    for r in range(_E_ROWS_PER_TILE):
      lrow = sid * _E_ROWS_PER_TILE + r
      copies.append(pltpu.async_copy(
          acc_s.at[pl.ds(lrow * _D_IN, _D_IN)],
          eout_hbm.at[e_row0 + lrow], sem))
    for r in range(_M_ROWS_PER_TILE):
      lrow = sid * _M_ROWS_PER_TILE + r
      copies.append(pltpu.async_copy(
          acc_s.at[pl.ds(_M_OFF + lrow * _D_CAT, _D_CAT)],
          mout_hbm.at[m_row0 + lrow], sem))
    for h in copies:
      h.wait()

  return dens


_densify_both = _make_densify_both()


def _tc_body(x_ref, e_ref, m_ref, o_ref):
  xb = x_ref[...]
  emb = lax.dot_general(xb, e_ref[...], (((1,), (1,)), ((), ())),
                        preferred_element_type=jnp.float32)
  cat = jnp.concatenate([xb, emb], axis=1)
  o_ref[...] = lax.dot_general(cat, m_ref[...], (((1,), (1,)), ((), ())),
                               preferred_element_type=jnp.float32)


_BB = 256


def _tc_forward(x, e_dense, m_dense):
  return pl.pallas_call(
      _tc_body,
      grid=(_B // _BB,),
      in_specs=[
          pl.BlockSpec((_BB, _D_IN), lambda i: (i, 0)),
          pl.BlockSpec((_N_EMB, _D_IN), lambda i: (0, 0)),
          pl.BlockSpec((_D_OUT, _D_CAT), lambda i: (0, 0)),
      ],
      out_specs=pl.BlockSpec((_BB, _D_OUT), lambda i: (i, 0)),
      out_shape=jax.ShapeDtypeStruct((_B, _D_OUT), jnp.float32),
  )(x, e_dense, m_dense)


def kernel(input, emb_vals, main_vals, emb_rows, emb_cols, main_rows, main_cols):
  e, m = _densify_both(
      emb_rows.astype(jnp.int32), emb_cols.astype(jnp.int32), emb_vals,
      main_rows.astype(jnp.int32), main_cols.astype(jnp.int32), main_vals)
  return _tc_forward(input, e, m)
